# Initial kernel scaffold; baseline (speedup 1.0000x reference)
import jax
import jax.numpy as jnp
from jax.experimental import pallas as pl


def kernel(edge_attr, edge_index, msg_emb, num_nodes, W_edge, b_edge, W0, b0, W4):
    raise NotImplementedError("write your pallas kernel here")


# speedup vs baseline: 2.4218x; 2.4218x over previous
import functools

import jax
import jax.numpy as jnp
from jax import lax
from jax.experimental import pallas as pl
from jax.experimental.pallas import tpu as pltpu
from jax.experimental.pallas import tpu_sc as plsc

N_NODES = 10000
N_EDGES = 320000
D_EDGE = 16
EMB = 128

NC = 2
NS = 16
NW = NC * NS
EPW = N_EDGES // NW
CH = 80
CPW = EPW // CH
NP = 10240
RPT = NP // NS

BE = 4000
BN = 2000


def _edge_body(attr_ref, msg_ref, w_ref, b_ref, o_ref):
    emb = jnp.dot(attr_ref[...], w_ref[...], preferred_element_type=jnp.float32)
    o_ref[...] = (emb + b_ref[...]) * msg_ref[...]


def _edge_stage(edge_attr, msg_emb, W_edge, b_edge):
    grid = (N_EDGES // BE,)
    return pl.pallas_call(
        _edge_body,
        grid=grid,
        in_specs=[
            pl.BlockSpec((BE, D_EDGE), lambda i: (i, 0)),
            pl.BlockSpec((BE, EMB), lambda i: (i, 0)),
            pl.BlockSpec((D_EDGE, EMB), lambda i: (0, 0)),
            pl.BlockSpec((1, EMB), lambda i: (0, 0)),
        ],
        out_specs=pl.BlockSpec((BE, EMB), lambda i: (i, 0)),
        out_shape=jax.ShapeDtypeStruct((N_EDGES, EMB), jnp.float32),
    )(edge_attr, msg_emb, W_edge, b_edge)


def _sc_body(x_hbm, ids_hbm, zeros_hbm, out_hbm, ids_row, xbuf, acc, sem):
    c = lax.axis_index("c")
    s = lax.axis_index("s")
    wid = s * NC + c
    pltpu.sync_copy(zeros_hbm, acc.at[pl.ds(s * RPT, RPT)])
    plsc.subcore_barrier()

    def body(j, carry):
        base = wid * EPW + j * CH
        pltpu.sync_copy(ids_hbm.at[pl.ds(base, CH)], ids_row)
        pltpu.sync_copy(x_hbm.at[pl.ds(base, CH)], xbuf)
        pltpu.sync_copy(xbuf, acc.at[ids_row], add=True)
        return carry

    lax.fori_loop(0, CPW, body, 0)
    plsc.subcore_barrier()
    pltpu.sync_copy(acc.at[pl.ds(s * RPT, RPT)],
                    out_hbm.at[c, pl.ds(s * RPT, RPT)])


@functools.cache
def _sc_scatter():
    return pl.kernel(
        _sc_body,
        out_type=jax.ShapeDtypeStruct((NC, NP, EMB), jnp.float32),
        mesh=plsc.VectorSubcoreMesh(
            core_axis_name="c", subcore_axis_name="s", num_cores=NC, num_subcores=NS
        ),
        scratch_types=[
            pltpu.VMEM((CH,), jnp.int32),
            pltpu.VMEM((CH, EMB), jnp.float32),
            pltpu.VMEM_SHARED((NP, EMB), jnp.float32),
            pltpu.SemaphoreType.DMA,
        ],
    )


def _mlp_body(p0_ref, p1_ref, w0_ref, b0_ref, w4_ref, o_ref):
    h = p0_ref[0] + p1_ref[0]
    w0 = w0_ref[...]
    b0 = b0_ref[...]
    x1 = jnp.maximum(jnp.dot(h, w0, preferred_element_type=jnp.float32) + b0, 0.0)
    x2 = jnp.maximum(jnp.dot(x1, w0, preferred_element_type=jnp.float32) + b0, 0.0)
    x3 = jnp.maximum(jnp.dot(x2, w0, preferred_element_type=jnp.float32) + b0, 0.0)
    o_ref[...] = jnp.dot(x3, w4_ref[...], preferred_element_type=jnp.float32)


def _mlp_stage(partials, W0, b0, W4):
    nblk = N_NODES // BN
    return pl.pallas_call(
        _mlp_body,
        grid=(nblk,),
        in_specs=[
            pl.BlockSpec((1, BN, EMB), lambda i: (0, i, 0)),
            pl.BlockSpec((1, BN, EMB), lambda i: (1, i, 0)),
            pl.BlockSpec((EMB, EMB), lambda i: (0, 0)),
            pl.BlockSpec((1, EMB), lambda i: (0, 0)),
            pl.BlockSpec((EMB, EMB), lambda i: (0, 0)),
        ],
        out_specs=pl.BlockSpec((BN, EMB), lambda i: (i, 0)),
        out_shape=jax.ShapeDtypeStruct((N_NODES, EMB), jnp.float32),
    )(partials, partials, W0, b0, W4)


def kernel(edge_attr, edge_index, msg_emb, num_nodes, W_edge, b_edge, W0, b0, W4):
    x = _edge_stage(edge_attr, msg_emb, W_edge, b_edge.reshape(1, EMB))
    ids = edge_index[1]
    zeros = jnp.zeros((RPT, EMB), dtype=jnp.float32)
    partials = _sc_scatter()(x, ids, zeros)
    return _mlp_stage(partials, W0, b0.reshape(1, EMB), W4)


# speedup vs baseline: 3.4725x; 1.4338x over previous
import functools

import jax
import jax.numpy as jnp
from jax import lax
from jax.experimental import pallas as pl
from jax.experimental.pallas import tpu as pltpu
from jax.experimental.pallas import tpu_sc as plsc

N_NODES = 10000
N_EDGES = 320000
D_EDGE = 16
EMB = 128

NC = 2
NS = 16
NW = NC * NS
EPW = N_EDGES // NW
CH = 80
CPW = EPW // CH
NP = 10240
RPT = NP // NS

BE = 4000
BN = 2000


def _edge_body(attr_ref, msg_ref, w_ref, b_ref, o_ref):
    emb = jnp.dot(attr_ref[...], w_ref[...], preferred_element_type=jnp.float32)
    o_ref[...] = (emb + b_ref[...]) * msg_ref[...]


def _edge_stage(edge_attr, msg_emb, W_edge, b_edge):
    grid = (N_EDGES // BE,)
    return pl.pallas_call(
        _edge_body,
        grid=grid,
        in_specs=[
            pl.BlockSpec((BE, D_EDGE), lambda i: (i, 0)),
            pl.BlockSpec((BE, EMB), lambda i: (i, 0)),
            pl.BlockSpec((D_EDGE, EMB), lambda i: (0, 0)),
            pl.BlockSpec((1, EMB), lambda i: (0, 0)),
        ],
        out_specs=pl.BlockSpec((BE, EMB), lambda i: (i, 0)),
        out_shape=jax.ShapeDtypeStruct((N_EDGES, EMB), jnp.float32),
    )(edge_attr, msg_emb, W_edge, b_edge)


NB = 3
NFULL = CPW // NB
NTAIL = CPW - NFULL * NB


def _sc_body(x_hbm, ids_hbm, zeros_hbm, out_hbm, xbs, idbs, acc, xsems, isems):
    c = lax.axis_index("c")
    s = lax.axis_index("s")
    wid = s * NC + c
    base = wid * EPW
    for t in range(NB):
        pltpu.async_copy(x_hbm.at[pl.ds(base + t * CH, CH)], xbs[t], xsems[t])
        pltpu.async_copy(ids_hbm.at[pl.ds(base + t * CH, CH)], idbs[t], isems[t])
    pltpu.sync_copy(zeros_hbm, acc.at[pl.ds(s * RPT, RPT)])
    plsc.subcore_barrier()

    def body(i, carry):
        for t in range(NB):
            chunk = i * NB + t
            pltpu.make_async_copy(
                x_hbm.at[pl.ds(base + chunk * CH, CH)], xbs[t], xsems[t]).wait()
            pltpu.make_async_copy(
                ids_hbm.at[pl.ds(base + chunk * CH, CH)], idbs[t], isems[t]).wait()
            pltpu.sync_copy(xbs[t], acc.at[idbs[t]], add=True)

            @pl.when(chunk + NB < CPW)
            def _():
                nxt = base + (chunk + NB) * CH
                pltpu.async_copy(x_hbm.at[pl.ds(nxt, CH)], xbs[t], xsems[t])
                pltpu.async_copy(ids_hbm.at[pl.ds(nxt, CH)], idbs[t], isems[t])

        return carry

    lax.fori_loop(0, NFULL, body, 0)
    for t in range(NTAIL):
        chunk = NFULL * NB + t
        pltpu.make_async_copy(
            x_hbm.at[pl.ds(base + chunk * CH, CH)], xbs[t], xsems[t]).wait()
        pltpu.make_async_copy(
            ids_hbm.at[pl.ds(base + chunk * CH, CH)], idbs[t], isems[t]).wait()
        pltpu.sync_copy(xbs[t], acc.at[idbs[t]], add=True)
    plsc.subcore_barrier()
    pltpu.sync_copy(acc.at[pl.ds(s * RPT, RPT)],
                    out_hbm.at[c, pl.ds(s * RPT, RPT)])


@functools.cache
def _sc_scatter():
    return pl.kernel(
        _sc_body,
        out_type=jax.ShapeDtypeStruct((NC, NP, EMB), jnp.float32),
        mesh=plsc.VectorSubcoreMesh(
            core_axis_name="c", subcore_axis_name="s", num_cores=NC, num_subcores=NS
        ),
        scratch_types=[
            tuple(pltpu.VMEM((CH, EMB), jnp.float32) for _ in range(NB)),
            tuple(pltpu.VMEM((CH,), jnp.int32) for _ in range(NB)),
            pltpu.VMEM_SHARED((NP, EMB), jnp.float32),
            tuple(pltpu.SemaphoreType.DMA for _ in range(NB)),
            tuple(pltpu.SemaphoreType.DMA for _ in range(NB)),
        ],
    )


def _mlp_body(p0_ref, p1_ref, w0_ref, b0_ref, w4_ref, o_ref):
    h = p0_ref[0] + p1_ref[0]
    w0 = w0_ref[...]
    b0 = b0_ref[...]
    x1 = jnp.maximum(jnp.dot(h, w0, preferred_element_type=jnp.float32) + b0, 0.0)
    x2 = jnp.maximum(jnp.dot(x1, w0, preferred_element_type=jnp.float32) + b0, 0.0)
    x3 = jnp.maximum(jnp.dot(x2, w0, preferred_element_type=jnp.float32) + b0, 0.0)
    o_ref[...] = jnp.dot(x3, w4_ref[...], preferred_element_type=jnp.float32)


def _mlp_stage(partials, W0, b0, W4):
    nblk = N_NODES // BN
    return pl.pallas_call(
        _mlp_body,
        grid=(nblk,),
        in_specs=[
            pl.BlockSpec((1, BN, EMB), lambda i: (0, i, 0)),
            pl.BlockSpec((1, BN, EMB), lambda i: (1, i, 0)),
            pl.BlockSpec((EMB, EMB), lambda i: (0, 0)),
            pl.BlockSpec((1, EMB), lambda i: (0, 0)),
            pl.BlockSpec((EMB, EMB), lambda i: (0, 0)),
        ],
        out_specs=pl.BlockSpec((BN, EMB), lambda i: (i, 0)),
        out_shape=jax.ShapeDtypeStruct((N_NODES, EMB), jnp.float32),
    )(partials, partials, W0, b0, W4)


def kernel(edge_attr, edge_index, msg_emb, num_nodes, W_edge, b_edge, W0, b0, W4):
    x = _edge_stage(edge_attr, msg_emb, W_edge, b_edge.reshape(1, EMB))
    zeros = jnp.zeros((RPT, EMB), dtype=jnp.float32)
    partials = _sc_scatter()(x, edge_index[1], zeros)
    return _mlp_stage(partials, W0, b0.reshape(1, EMB), W4)


# speedup vs baseline: 5.1768x; 1.4908x over previous
import functools

import jax
import jax.numpy as jnp
from jax import lax
from jax.experimental import pallas as pl
from jax.experimental.pallas import tpu as pltpu
from jax.experimental.pallas import tpu_sc as plsc

N_NODES = 10000
N_EDGES = 320000
D_EDGE = 16
EMB = 128

NC = 2
NS = 16
NW = NC * NS
EPW = N_EDGES // NW
CH = 80
CPW = EPW // CH
NP = 10240
RPT = NP // NS

BE = 6400
BN = 2000


def _edge_body(attr_ref, msg_ref, w_ref, b_ref, o_ref):
    emb = lax.dot_general(attr_ref[...], w_ref[...],
                          dimension_numbers=(((0,), (0,)), ((), ())),
                          preferred_element_type=jnp.float32)
    o_ref[...] = (emb + b_ref[...]) * msg_ref[...]


def _edge_stage(edge_attr_t, msg_emb, W_edge, b_edge):
    grid = (N_EDGES // BE,)
    return pl.pallas_call(
        _edge_body,
        grid=grid,
        in_specs=[
            pl.BlockSpec((D_EDGE, BE), lambda i: (0, i)),
            pl.BlockSpec((BE, EMB), lambda i: (i, 0)),
            pl.BlockSpec((D_EDGE, EMB), lambda i: (0, 0)),
            pl.BlockSpec((1, EMB), lambda i: (0, 0)),
        ],
        out_specs=pl.BlockSpec((BE, EMB), lambda i: (i, 0)),
        out_shape=jax.ShapeDtypeStruct((N_EDGES, EMB), jnp.float32),
    )(edge_attr_t, msg_emb, W_edge, b_edge)


NB = 3
NFULL = CPW // NB
NTAIL = CPW - NFULL * NB


def _sc_body(x_hbm, ids_hbm, zeros_hbm, out_hbm, xbs, idbs, acc, xsems, isems):
    c = lax.axis_index("c")
    s = lax.axis_index("s")
    wid = s * NC + c
    base = wid * EPW
    for t in range(NB):
        pltpu.async_copy(x_hbm.at[pl.ds(base + t * CH, CH)], xbs[t], xsems[t])
        pltpu.async_copy(ids_hbm.at[pl.ds(base + t * CH, CH)], idbs[t], isems[t])
    pltpu.sync_copy(zeros_hbm, acc.at[pl.ds(s * RPT, RPT)])
    plsc.subcore_barrier()

    def body(i, carry):
        for t in range(NB):
            chunk = i * NB + t
            pltpu.make_async_copy(
                x_hbm.at[pl.ds(base + chunk * CH, CH)], xbs[t], xsems[t]).wait()
            pltpu.make_async_copy(
                ids_hbm.at[pl.ds(base + chunk * CH, CH)], idbs[t], isems[t]).wait()
            pltpu.sync_copy(xbs[t], acc.at[idbs[t]], add=True)

            @pl.when(chunk + NB < CPW)
            def _():
                nxt = base + (chunk + NB) * CH
                pltpu.async_copy(x_hbm.at[pl.ds(nxt, CH)], xbs[t], xsems[t])
                pltpu.async_copy(ids_hbm.at[pl.ds(nxt, CH)], idbs[t], isems[t])

        return carry

    lax.fori_loop(0, NFULL, body, 0)
    for t in range(NTAIL):
        chunk = NFULL * NB + t
        pltpu.make_async_copy(
            x_hbm.at[pl.ds(base + chunk * CH, CH)], xbs[t], xsems[t]).wait()
        pltpu.make_async_copy(
            ids_hbm.at[pl.ds(base + chunk * CH, CH)], idbs[t], isems[t]).wait()
        pltpu.sync_copy(xbs[t], acc.at[idbs[t]], add=True)
    plsc.subcore_barrier()
    pltpu.sync_copy(acc.at[pl.ds(s * RPT, RPT)],
                    out_hbm.at[c, pl.ds(s * RPT, RPT)])


@functools.cache
def _sc_scatter():
    return pl.kernel(
        _sc_body,
        out_type=jax.ShapeDtypeStruct((NC, NP, EMB), jnp.float32),
        mesh=plsc.VectorSubcoreMesh(
            core_axis_name="c", subcore_axis_name="s", num_cores=NC, num_subcores=NS
        ),
        scratch_types=[
            tuple(pltpu.VMEM((CH, EMB), jnp.float32) for _ in range(NB)),
            tuple(pltpu.VMEM((CH,), jnp.int32) for _ in range(NB)),
            pltpu.VMEM_SHARED((NP, EMB), jnp.float32),
            tuple(pltpu.SemaphoreType.DMA for _ in range(NB)),
            tuple(pltpu.SemaphoreType.DMA for _ in range(NB)),
        ],
    )


def _mlp_body(p0_ref, p1_ref, w0_ref, b0_ref, w4_ref, o_ref):
    h = p0_ref[0] + p1_ref[0]
    w0 = w0_ref[...]
    b0 = b0_ref[...]
    x1 = jnp.maximum(jnp.dot(h, w0, preferred_element_type=jnp.float32) + b0, 0.0)
    x2 = jnp.maximum(jnp.dot(x1, w0, preferred_element_type=jnp.float32) + b0, 0.0)
    x3 = jnp.maximum(jnp.dot(x2, w0, preferred_element_type=jnp.float32) + b0, 0.0)
    o_ref[...] = jnp.dot(x3, w4_ref[...], preferred_element_type=jnp.float32)


def _mlp_stage(partials, W0, b0, W4):
    nblk = N_NODES // BN
    return pl.pallas_call(
        _mlp_body,
        grid=(nblk,),
        in_specs=[
            pl.BlockSpec((1, BN, EMB), lambda i: (0, i, 0)),
            pl.BlockSpec((1, BN, EMB), lambda i: (1, i, 0)),
            pl.BlockSpec((EMB, EMB), lambda i: (0, 0)),
            pl.BlockSpec((1, EMB), lambda i: (0, 0)),
            pl.BlockSpec((EMB, EMB), lambda i: (0, 0)),
        ],
        out_specs=pl.BlockSpec((BN, EMB), lambda i: (i, 0)),
        out_shape=jax.ShapeDtypeStruct((N_NODES, EMB), jnp.float32),
    )(partials, partials, W0, b0, W4)


def kernel(edge_attr, edge_index, msg_emb, num_nodes, W_edge, b_edge, W0, b0, W4):
    x = _edge_stage(edge_attr.T, msg_emb, W_edge, b_edge.reshape(1, EMB))
    zeros = jnp.zeros((RPT, EMB), dtype=jnp.float32)
    partials = _sc_scatter()(x, edge_index[1], zeros)
    return _mlp_stage(partials, W0, b0.reshape(1, EMB), W4)
